# Initial kernel scaffold; baseline (speedup 1.0000x reference)
import jax
import jax.numpy as jnp
from jax.experimental import pallas as pl


def kernel(x, edge_index, batch, params):
    raise NotImplementedError("write your pallas kernel here")


# speedup vs baseline: 5.8632x; 5.8632x over previous
import functools

import jax
import jax.numpy as jnp
from jax import lax
from jax.experimental import pallas as pl
from jax.experimental.pallas import tpu as pltpu
from jax.experimental.pallas import tpu_sc as plsc

_NC = 2
_NS = 16
_K = 80
_G = 128


@functools.partial(jax.jit, static_argnames=("np_", "hd", "nch"))
def _seg_sum_sc(h, src3, dst3, zeros, *, np_, hd, nch):
  rows_per_tile = np_ // _NS

  def body(h_hbm, src_hbm, dst_hbm, zeros_hbm, out_hbm,
           src_v, dst_v, rows_v, agg_sh, sem):
    cid = lax.axis_index("c")
    sid = lax.axis_index("s")
    wid = cid * _NS + sid
    row0 = sid * rows_per_tile
    pltpu.sync_copy(zeros_hbm.at[pl.ds(row0, rows_per_tile)],
                    agg_sh.at[pl.ds(row0, rows_per_tile)])
    plsc.subcore_barrier()

    def step(c, _):
      pltpu.sync_copy(src_hbm.at[wid, c], src_v)
      pltpu.sync_copy(dst_hbm.at[wid, c], dst_v)
      pltpu.async_copy(h_hbm.at[src_v], rows_v, sem).wait()
      pltpu.sync_copy(rows_v, agg_sh.at[dst_v], add=True)
      return 0

    lax.fori_loop(0, nch, step, 0)
    plsc.subcore_barrier()
    pltpu.sync_copy(agg_sh.at[pl.ds(row0, rows_per_tile)],
                    out_hbm.at[cid, pl.ds(row0, rows_per_tile)])

  fn = pl.kernel(
      body,
      out_type=jax.ShapeDtypeStruct((_NC, np_, hd), jnp.float32),
      mesh=plsc.VectorSubcoreMesh(core_axis_name="c", subcore_axis_name="s",
                                  num_cores=_NC, num_subcores=_NS),
      scratch_types=[
          pltpu.VMEM((_K,), jnp.int32),
          pltpu.VMEM((_K,), jnp.int32),
          pltpu.VMEM((_K, hd), jnp.float32),
          pltpu.VMEM_SHARED((np_, hd), jnp.float32),
          pltpu.SemaphoreType.DMA,
      ],
      compiler_params=pltpu.CompilerParams(use_tc_tiling_on_sc=False),
  )
  return fn(h, src3, dst3, zeros)


def _layer_body(h_ref, s_ref, w1_ref, b1_ref, w2_ref, b2_ref, g_ref, bb_ref,
                o_ref):
  n = h_ref.shape[0]
  z = h_ref[...] + s_ref[0, :n] + s_ref[1, :n]
  z1 = jnp.maximum(jnp.dot(z, w1_ref[...],
                           preferred_element_type=jnp.float32) + b1_ref[...],
                   0.0)
  z2 = jnp.dot(z1, w2_ref[...],
               preferred_element_type=jnp.float32) + b2_ref[...]
  r = jnp.maximum(z2, 0.0)
  mean = jnp.mean(r, axis=0, keepdims=True)
  var = jnp.mean((r - mean) ** 2, axis=0, keepdims=True)
  o_ref[...] = (r - mean) / jnp.sqrt(var + 1e-5) * g_ref[...] + bb_ref[...]


def _layer(h, s, w1, b1, w2, b2, g, bb):
  return pl.pallas_call(
      _layer_body,
      out_shape=jax.ShapeDtypeStruct((h.shape[0], w2.shape[1]), jnp.float32),
  )(h, s, w1, b1, w2, b2, g, bb)


def _final_body(h_ref, batch_ref, f1w_ref, f1b_ref, f2w_ref, f2b_ref, o_ref):
  n = h_ref.shape[0]
  onehot = (batch_ref[...] ==
            lax.broadcasted_iota(jnp.int32, (n, _G), 1)).astype(jnp.float32)
  pooled = lax.dot_general(onehot, h_ref[...], (((0,), (0,)), ((), ())),
                           precision=lax.Precision.HIGHEST,
                           preferred_element_type=jnp.float32)
  z = jnp.maximum(jnp.dot(pooled, f1w_ref[...],
                          preferred_element_type=jnp.float32) + f1b_ref[...],
                  0.0)
  z = jnp.dot(z, f2w_ref[...],
              preferred_element_type=jnp.float32) + f2b_ref[...]
  m = jnp.max(z, axis=-1, keepdims=True)
  lse = jnp.log(jnp.sum(jnp.exp(z - m), axis=-1, keepdims=True)) + m
  o_ref[...] = z - lse


def _final(h, batch2, f1w, f1b, f2w, f2b):
  return pl.pallas_call(
      _final_body,
      out_shape=jax.ShapeDtypeStruct((_G, f2w.shape[1]), jnp.float32),
  )(h, batch2, f1w, f1b, f2w, f2b)


def kernel(x, edge_index, batch, params):
  n, d = x.shape
  e = edge_index.shape[1]
  h = params["conv1_w1"].shape[1]
  nw = _NC * _NS
  nch = e // (nw * _K)
  np_ = (n + _NS * 8 - 1) // (_NS * 8) * (_NS * 8)
  src3 = edge_index[0].reshape(nw, nch, _K)
  dst3 = edge_index[1].reshape(nw, nch, _K)
  zeros_d = jnp.zeros((np_, d), jnp.float32)
  zeros_h = jnp.zeros((np_, h), jnp.float32)
  batch2 = batch.reshape(n, 1)
  row = lambda v: v.reshape(1, -1)

  hcur = x
  for l in range(1, 6):
    zeros = zeros_d if hcur.shape[1] == d else zeros_h
    s = _seg_sum_sc(hcur, src3, dst3, zeros, np_=np_, hd=hcur.shape[1],
                    nch=nch)
    hcur = _layer(hcur, s, params[f"conv{l}_w1"], row(params[f"conv{l}_b1"]),
                  params[f"conv{l}_w2"], row(params[f"conv{l}_b2"]),
                  row(params[f"bn{l}_g"]), row(params[f"bn{l}_b"]))
  return _final(hcur, batch2, params["fc1_w"], row(params["fc1_b"]),
                params["fc2_w"], row(params["fc2_b"]))


# speedup vs baseline: 6.6301x; 1.1308x over previous
import functools

import jax
import jax.numpy as jnp
from jax import lax
from jax.experimental import pallas as pl
from jax.experimental.pallas import tpu as pltpu
from jax.experimental.pallas import tpu_sc as plsc

_NC = 2
_NS = 16
_K = 128
_G = 128


@functools.partial(jax.jit, static_argnames=("np_", "hd", "nch"))
def _seg_sum_sc(h, idx4, zeros, *, np_, hd, nch):
  rows_per_tile = np_ // _NS

  def body(h_hbm, idx_hbm, zeros_hbm, out_hbm,
           idx_a, idx_b, rows_a, rows_b, agg_sh, sem_a, sem_b):
    cid = lax.axis_index("c")
    sid = lax.axis_index("s")
    wid = cid * _NS + sid
    row0 = sid * rows_per_tile
    pltpu.sync_copy(zeros_hbm.at[pl.ds(row0, rows_per_tile)],
                    agg_sh.at[pl.ds(row0, rows_per_tile)])

    bufs = ((idx_a, rows_a, sem_a), (idx_b, rows_b, sem_b))

    def fire(idx_v, rows_v, sem, c):
      pltpu.sync_copy(idx_hbm.at[wid, c], idx_v)
      pltpu.async_copy(h_hbm.at[idx_v.at[0]], rows_v, sem)

    def drain(idx_v, rows_v, sem):
      pltpu.make_async_copy(h_hbm.at[idx_v.at[0]], rows_v, sem).wait()
      pltpu.sync_copy(rows_v, agg_sh.at[idx_v.at[1]], add=True)

    plsc.subcore_barrier()
    for b, (idx_v, rows_v, sem) in enumerate(bufs):
      fire(idx_v, rows_v, sem, b)

    def step(i, _):
      c = 2 * i
      for b, (idx_v, rows_v, sem) in enumerate(bufs):
        drain(idx_v, rows_v, sem)
        fire(idx_v, rows_v, sem, c + 2 + b)
      return 0

    lax.fori_loop(0, nch // 2 - 1, step, 0)
    for idx_v, rows_v, sem in bufs:
      drain(idx_v, rows_v, sem)
    plsc.subcore_barrier()
    pltpu.sync_copy(agg_sh.at[pl.ds(row0, rows_per_tile)],
                    out_hbm.at[cid, pl.ds(row0, rows_per_tile)])

  fn = pl.kernel(
      body,
      out_type=jax.ShapeDtypeStruct((_NC, np_, hd), jnp.float32),
      mesh=plsc.VectorSubcoreMesh(core_axis_name="c", subcore_axis_name="s",
                                  num_cores=_NC, num_subcores=_NS),
      scratch_types=[
          pltpu.VMEM((2, _K), jnp.int32),
          pltpu.VMEM((2, _K), jnp.int32),
          pltpu.VMEM((_K, hd), jnp.float32),
          pltpu.VMEM((_K, hd), jnp.float32),
          pltpu.VMEM_SHARED((np_, hd), jnp.float32),
          pltpu.SemaphoreType.DMA,
          pltpu.SemaphoreType.DMA,
      ],
      compiler_params=pltpu.CompilerParams(use_tc_tiling_on_sc=False),
  )
  return fn(h, idx4, zeros)


def _layer_body(h_ref, s_ref, w1_ref, b1_ref, w2_ref, b2_ref, g_ref, bb_ref,
                o_ref):
  n = h_ref.shape[0]
  z = h_ref[...] + s_ref[0, :n] + s_ref[1, :n]
  z1 = jnp.maximum(jnp.dot(z, w1_ref[...],
                           preferred_element_type=jnp.float32) + b1_ref[...],
                   0.0)
  z2 = jnp.dot(z1, w2_ref[...],
               preferred_element_type=jnp.float32) + b2_ref[...]
  r = jnp.maximum(z2, 0.0)
  mean = jnp.mean(r, axis=0, keepdims=True)
  var = jnp.mean((r - mean) ** 2, axis=0, keepdims=True)
  o_ref[...] = (r - mean) / jnp.sqrt(var + 1e-5) * g_ref[...] + bb_ref[...]


def _layer(h, s, w1, b1, w2, b2, g, bb):
  return pl.pallas_call(
      _layer_body,
      out_shape=jax.ShapeDtypeStruct((h.shape[0], w2.shape[1]), jnp.float32),
  )(h, s, w1, b1, w2, b2, g, bb)


def _final_body(h_ref, batch_ref, f1w_ref, f1b_ref, f2w_ref, f2b_ref, o_ref):
  n = h_ref.shape[0]
  onehot = (batch_ref[...] ==
            lax.broadcasted_iota(jnp.int32, (n, _G), 1)).astype(jnp.float32)
  pooled = lax.dot_general(onehot, h_ref[...], (((0,), (0,)), ((), ())),
                           precision=lax.Precision.HIGHEST,
                           preferred_element_type=jnp.float32)
  z = jnp.maximum(jnp.dot(pooled, f1w_ref[...],
                          preferred_element_type=jnp.float32) + f1b_ref[...],
                  0.0)
  z = jnp.dot(z, f2w_ref[...],
              preferred_element_type=jnp.float32) + f2b_ref[...]
  m = jnp.max(z, axis=-1, keepdims=True)
  lse = jnp.log(jnp.sum(jnp.exp(z - m), axis=-1, keepdims=True)) + m
  o_ref[...] = z - lse


def _final(h, batch2, f1w, f1b, f2w, f2b):
  return pl.pallas_call(
      _final_body,
      out_shape=jax.ShapeDtypeStruct((_G, f2w.shape[1]), jnp.float32),
  )(h, batch2, f1w, f1b, f2w, f2b)


def kernel(x, edge_index, batch, params):
  n, d = x.shape
  e = edge_index.shape[1]
  h = params["conv1_w1"].shape[1]
  nw = _NC * _NS
  np_ = (n + 1 + _NS * 8 - 1) // (_NS * 8) * (_NS * 8)
  nch = -(-e // (nw * _K))
  nch += nch % 2
  ep = nw * nch * _K
  pad = ep - e
  src_p = jnp.concatenate([edge_index[0], jnp.zeros((pad,), jnp.int32)])
  dst_p = jnp.concatenate([edge_index[1],
                           jnp.full((pad,), np_ - 1, jnp.int32)])
  idx4 = jnp.stack([src_p.reshape(nw, nch, _K),
                    dst_p.reshape(nw, nch, _K)], axis=2)
  zeros_d = jnp.zeros((np_, d), jnp.float32)
  zeros_h = jnp.zeros((np_, h), jnp.float32)
  batch2 = batch.reshape(n, 1)
  row = lambda v: v.reshape(1, -1)

  hcur = x
  for l in range(1, 6):
    zeros = zeros_d if hcur.shape[1] == d else zeros_h
    s = _seg_sum_sc(hcur, idx4, zeros, np_=np_, hd=hcur.shape[1], nch=nch)
    hcur = _layer(hcur, s, params[f"conv{l}_w1"], row(params[f"conv{l}_b1"]),
                  params[f"conv{l}_w2"], row(params[f"conv{l}_b2"]),
                  row(params[f"bn{l}_g"]), row(params[f"bn{l}_b"]))
  return _final(hcur, batch2, params["fc1_w"], row(params["fc1_b"]),
                params["fc2_w"], row(params["fc2_b"]))
